# baseline (speedup 1.0000x reference)
import functools

import jax
import jax.numpy as jnp
from jax import lax
from jax.experimental import pallas as pl
from jax.experimental.pallas import tpu as pltpu
from jax.experimental.pallas import tpu_sc as plsc

V = 100000
D = 64
B = 4096
F = 26

NC = 2
NS = 16
NW = NC * NS

LANES = 128
BLK = 2
BW = BLK * LANES
NBLK = (V // LANES) // BLK
REST = V - NBLK * BW
KMAX = (NBLK + NW - 1) // NW
_mesh = plsc.VectorSubcoreMesh(
    core_axis_name="c", subcore_axis_name="s", num_cores=NC, num_subcores=NS
)


@functools.partial(
    pl.kernel,
    out_type=jax.ShapeDtypeStruct((V * D // 2,), jnp.int32),
    mesh=_mesh,
    scratch_types=[
        pltpu.VMEM((D, BW), jnp.float32),
        pltpu.VMEM((D, BW), jnp.float32),
        pltpu.VMEM((BW * D // 2,), jnp.int32),
        pltpu.VMEM((BW * D // 2,), jnp.int32),
        pltpu.VMEM((REST * D // 2,), jnp.int32),
        pltpu.SemaphoreType.DMA,
        pltpu.SemaphoreType.DMA,
    ],
    compiler_params=pltpu.CompilerParams(
        use_tc_tiling_on_sc=True, needs_layout_passes=False
    ),
)
def _relayout(tt_hbm, tail_hbm, out_hbm, a0, a1, o0, o1, tbuf, sem_i, sem_o):
    wid = lax.axis_index("s") * NC + lax.axis_index("c")
    iota = lax.iota(jnp.int32, 16)
    d_idx = [iota + g * 16 for g in range(D // 16)]
    abufs, obufs = (a0, a1), (o0, o1)

    def blk_of(k):
        return jnp.minimum(wid + k * NW, NBLK - 1)

    def start_in(k):
        boff = pl.multiple_of(blk_of(k) * BW, BW)
        return [
            pltpu.async_copy(
                tt_hbm.at[pl.ds(r * 8, 8), pl.ds(boff, BW)],
                abufs[k % 2].at[pl.ds(r * 8, 8)],
                sem_i,
            )
            for r in range(D // 8)
        ]

    in_cp = [start_in(0), None]
    out_cp = [None, None]
    for k in range(KMAX):
        abuf, obuf = abufs[k % 2], obufs[k % 2]
        for cp in in_cp[k % 2]:
            cp.wait()
        if k + 1 < KMAX:
            in_cp[(k + 1) % 2] = start_in(k + 1)
        if out_cp[k % 2] is not None:
            out_cp[k % 2].wait()

        @plsc.parallel_loop(0, BW, step=1, unroll=8)
        def xpose(jm, abuf=abuf, obuf=obuf):
            j_idx = jnp.broadcast_to(jm, (16,))
            g0 = plsc.load_gather(abuf, [d_idx[0], j_idx])
            g1 = plsc.load_gather(abuf, [d_idx[1], j_idx])
            g2 = plsc.load_gather(abuf, [d_idx[2], j_idx])
            g3 = plsc.load_gather(abuf, [d_idx[3], j_idx])
            u0, u1 = plsc.bitcast(g0, jnp.uint32), plsc.bitcast(g1, jnp.uint32)
            u2, u3 = plsc.bitcast(g2, jnp.uint32), plsc.bitcast(g3, jnp.uint32)
            hi_mask = jnp.uint32(0xFFFF0000)
            w01 = (u0 >> 16) | (u1 & hi_mask)
            w23 = (u2 >> 16) | (u3 & hi_mask)
            off = pl.multiple_of(jm * (D // 2), D // 2)
            obuf[pl.ds(off, 16)] = plsc.bitcast(w01, jnp.int32)
            obuf[pl.ds(off + 16, 16)] = plsc.bitcast(w23, jnp.int32)

        oo = pl.multiple_of(blk_of(k) * (BW * D // 2), 8)
        out_cp[k % 2] = pltpu.async_copy(
            obuf, out_hbm.at[pl.ds(oo, BW * D // 2)], sem_o
        )

    out_cp[(KMAX - 2) % 2].wait()
    out_cp[(KMAX - 1) % 2].wait()

    @pl.when(wid == NW - 1)
    def _tail():
        pltpu.sync_copy(tail_hbm, tbuf)
        pltpu.sync_copy(tbuf, out_hbm.at[pl.ds(NBLK * BW * D // 2, REST * D // 2)])


BPW = B // NW
MC = 8
MB = BPW // MC
ROWS = MB * F


@functools.partial(
    pl.kernel,
    out_type=jax.ShapeDtypeStruct((B, D), jnp.float32),
    mesh=_mesh,
    scratch_types=[
        pltpu.VMEM((BPW, F), jnp.int32),
        pltpu.VMEM((ROWS, D // 2), jnp.int32),
        pltpu.VMEM((ROWS, D // 2), jnp.int32),
        pltpu.VMEM((BPW, D), jnp.float32),
        pltpu.SemaphoreType.DMA,
    ],
    compiler_params=pltpu.CompilerParams(
        use_tc_tiling_on_sc=False, needs_layout_passes=False
    ),
)
def _field_embed(x_hbm, table_hbm, out_hbm, idx_v, buf0, buf1, out_v, sem):
    wid = lax.axis_index("s") * NC + lax.axis_index("c")
    pltpu.sync_copy(x_hbm.at[pl.ds(wid * BPW, BPW)], idx_v)

    bufs = (buf0, buf1)

    def start_gather(m, buf):
        return [
            pltpu.async_copy(
                table_hbm.at[idx_v.at[m * MB + j]],
                buf.at[pl.ds(j * F, F)],
                sem,
            )
            for j in range(MB)
        ]

    copies = start_gather(0, bufs[0])
    for m in range(MC):
        buf = bufs[m % 2]
        for cp in copies:
            cp.wait()
        if m + 1 < MC:
            copies = start_gather(m + 1, bufs[(m + 1) % 2])

        def pool_row(b, _, buf=buf, m=m):
            base = b * F
            acc = [None] * 4
            for f in range(F):
                lo = plsc.bitcast(buf[base + f, pl.ds(0, 16)], jnp.bfloat16)
                hi = plsc.bitcast(buf[base + f, pl.ds(16, 16)], jnp.bfloat16)
                p01 = plsc.unpack(lo, format=plsc.PackFormat.INTERLEAVED)
                p23 = plsc.unpack(hi, format=plsc.PackFormat.INTERLEAVED)
                parts = (p01[0], p01[1], p23[0], p23[1])
                for d in range(4):
                    acc[d] = parts[d] if f == 0 else acc[d] + parts[d]
            row = m * MB + b
            for d in range(4):
                out_v[row, pl.ds(d * 16, 16)] = acc[d]
            return 0

        lax.fori_loop(0, MB, pool_row, 0)

    pltpu.sync_copy(out_v, out_hbm.at[pl.ds(wid * BPW, BPW)])


_PACK_PERM = [
    half + off
    for half in (0, 32)
    for i in range(16)
    for off in (i, i + 16)
]


def kernel(x, table):
    tail_bf = (
        table[NBLK * BW :, :]
        .astype(jnp.bfloat16)[:, jnp.array(_PACK_PERM)]
        .reshape(REST, D // 2, 2)
    )
    tail = jax.lax.bitcast_convert_type(tail_bf, jnp.int32).reshape(-1)
    t_flat = _relayout(table.T, tail)
    t2d = t_flat.reshape(V, D // 2)
    return _field_embed(x.astype(jnp.int32), t2d)


# speedup vs baseline: 1.0828x; 1.0828x over previous
import functools

import jax
import jax.numpy as jnp
from jax import lax
from jax.experimental import pallas as pl
from jax.experimental.pallas import tpu as pltpu
from jax.experimental.pallas import tpu_sc as plsc

V = 100000
D = 64
B = 4096
F = 26

NC = 2
NS = 16
NW = NC * NS
BPW = B // NW
MC = 8
MB = BPW // MC
ROWS = MB * F

_mesh = plsc.VectorSubcoreMesh(
    core_axis_name="c", subcore_axis_name="s", num_cores=NC, num_subcores=NS
)


@functools.partial(
    pl.kernel,
    out_type=jax.ShapeDtypeStruct((B, D), jnp.float32),
    mesh=_mesh,
    scratch_types=[
        pltpu.VMEM((BPW, F), jnp.int32),
        pltpu.VMEM((ROWS, D), jnp.bfloat16),
        pltpu.VMEM((ROWS, D), jnp.bfloat16),
        pltpu.VMEM((BPW, D), jnp.float32),
        pltpu.SemaphoreType.DMA,
    ],
    compiler_params=pltpu.CompilerParams(
        use_tc_tiling_on_sc=False, needs_layout_passes=False
    ),
)
def _field_embed(x_hbm, table_hbm, out_hbm, idx_v, buf0, buf1, out_v, sem):
    wid = lax.axis_index("s") * NC + lax.axis_index("c")
    pltpu.sync_copy(x_hbm.at[pl.ds(wid * BPW, BPW)], idx_v)

    bufs = (buf0, buf1)

    def start_gather(m, buf):
        return [
            pltpu.async_copy(
                table_hbm.at[idx_v.at[m * MB + j]],
                buf.at[pl.ds(j * F, F)],
                sem,
            )
            for j in range(MB)
        ]

    copies = start_gather(0, bufs[0])
    for m in range(MC):
        buf = bufs[m % 2]
        for cp in copies:
            cp.wait()
        if m + 1 < MC:
            copies = start_gather(m + 1, bufs[(m + 1) % 2])

        def pool_row(b, _, buf=buf, m=m):
            base = b * F
            acc = [None] * 4
            for f in range(F):
                lo = buf[base + f, pl.ds(0, 32)]
                hi = buf[base + f, pl.ds(32, 32)]
                p01 = plsc.unpack(lo, format=plsc.PackFormat.INTERLEAVED)
                p23 = plsc.unpack(hi, format=plsc.PackFormat.INTERLEAVED)
                parts = (p01[0], p01[1], p23[0], p23[1])
                for d in range(4):
                    acc[d] = parts[d] if f == 0 else acc[d] + parts[d]
            row = m * MB + b
            for d in range(4):
                out_v[row, pl.ds(d * 16, 16)] = acc[d]
            return 0

        lax.fori_loop(0, MB, pool_row, 0)

    pltpu.sync_copy(out_v, out_hbm.at[pl.ds(wid * BPW, BPW)])


_ORDER = (
    list(range(0, 32, 2)) + list(range(1, 32, 2))
    + list(range(32, 64, 2)) + list(range(33, 64, 2))
)
_INV = [0] * D
for _i, _c in enumerate(_ORDER):
    _INV[_c] = _i


def kernel(x, table):
    tb = table.astype(jnp.bfloat16)
    out_p = _field_embed(x.astype(jnp.int32), tb)
    return out_p[:, jnp.array(_INV)]


# speedup vs baseline: 1.3798x; 1.2743x over previous
import functools

import jax
import jax.numpy as jnp
from jax import lax
from jax.experimental import pallas as pl
from jax.experimental.pallas import tpu as pltpu
from jax.experimental.pallas import tpu_sc as plsc

D = 64
B = 4096
F = 26

NC = 2
NS = 16
NW = NC * NS
BPW = B // NW
MC = 8
MB = BPW // MC
ROWS = MB * F

_mesh = plsc.VectorSubcoreMesh(
    core_axis_name="c", subcore_axis_name="s", num_cores=NC, num_subcores=NS
)


@functools.partial(
    pl.kernel,
    out_type=jax.ShapeDtypeStruct((B, D), jnp.float32),
    mesh=_mesh,
    scratch_types=[
        pltpu.VMEM((BPW, F), jnp.int32),
        pltpu.VMEM((ROWS, D), jnp.float32),
        pltpu.VMEM((ROWS, D), jnp.float32),
        pltpu.VMEM((BPW, D), jnp.float32),
        pltpu.SemaphoreType.DMA,
    ],
    compiler_params=pltpu.CompilerParams(use_tc_tiling_on_sc=False),
)
def _field_embed(x_hbm, table_hbm, out_hbm, idx_v, buf0, buf1, out_v, sem):
    wid = lax.axis_index("s") * NC + lax.axis_index("c")
    pltpu.sync_copy(x_hbm.at[pl.ds(wid * BPW, BPW)], idx_v)

    bufs = (buf0, buf1)

    def start_gather(m, buf):
        return [
            pltpu.async_copy(
                table_hbm.at[idx_v.at[m * MB + j]],
                buf.at[pl.ds(j * F, F)],
                sem,
            )
            for j in range(MB)
        ]

    copies = start_gather(0, bufs[0])
    for m in range(MC):
        buf = bufs[m % 2]
        for cp in copies:
            cp.wait()
        if m + 1 < MC:
            copies = start_gather(m + 1, bufs[(m + 1) % 2])

        def pool_row(b, _, buf=buf, m=m):
            base = b * F
            acc = [buf[base, pl.ds(d * 16, 16)] for d in range(D // 16)]
            for f in range(1, F):
                for d in range(D // 16):
                    acc[d] = acc[d] + buf[base + f, pl.ds(d * 16, 16)]
            row = m * MB + b
            for d in range(D // 16):
                out_v[row, pl.ds(d * 16, 16)] = acc[d]
            return 0

        lax.fori_loop(0, MB, pool_row, 0)

    pltpu.sync_copy(out_v, out_hbm.at[pl.ds(wid * BPW, BPW)])


def kernel(x, table):
    return _field_embed(x.astype(jnp.int32), table)


# speedup vs baseline: 1.4020x; 1.0160x over previous
import functools

import jax
import jax.numpy as jnp
from jax import lax
from jax.experimental import pallas as pl
from jax.experimental.pallas import tpu as pltpu
from jax.experimental.pallas import tpu_sc as plsc

NUM_EMB = 100000
D = 64
B = 4096
F = 26

NC = 2
NS = 16
NW = NC * NS
BPW = B // NW
MC = 8
MB = BPW // MC
ROWS = MB * F
NSUB = 4
SUBW = ROWS // NSUB

_mesh = plsc.VectorSubcoreMesh(
    core_axis_name="c", subcore_axis_name="s", num_cores=NC, num_subcores=NS
)


@functools.partial(
    pl.kernel,
    out_type=jax.ShapeDtypeStruct((B, D), jnp.float32),
    mesh=_mesh,
    scratch_types=[
        pltpu.VMEM((MC, NSUB, SUBW), jnp.int32),
        pltpu.VMEM((ROWS, D), jnp.float32),
        pltpu.VMEM((ROWS, D), jnp.float32),
        pltpu.VMEM((BPW, D), jnp.float32),
        pltpu.SemaphoreType.DMA,
    ],
    compiler_params=pltpu.CompilerParams(use_tc_tiling_on_sc=False),
)
def _field_embed(x_hbm, table_hbm, out_hbm, idx_v, buf0, buf1, out_v, sem):
    wid = lax.axis_index("s") * NC + lax.axis_index("c")
    pltpu.sync_copy(x_hbm.at[wid], idx_v)

    bufs = (buf0, buf1)

    def start_gather(m, buf):
        return [
            pltpu.async_copy(
                table_hbm.at[idx_v.at[m, sub]],
                buf.at[pl.ds(sub * SUBW, SUBW)],
                sem,
            )
            for sub in range(NSUB)
        ]

    copies = start_gather(0, bufs[0])
    for m in range(MC):
        buf = bufs[m % 2]
        for cp in copies:
            cp.wait()
        if m + 1 < MC:
            copies = start_gather(m + 1, bufs[(m + 1) % 2])

        def pool_row(b, _, buf=buf, m=m):
            base = b * F
            acc = [buf[base, pl.ds(d * 16, 16)] for d in range(D // 16)]
            for f in range(1, F):
                for d in range(D // 16):
                    acc[d] = acc[d] + buf[base + f, pl.ds(d * 16, 16)]
            row = m * MB + b
            for d in range(D // 16):
                out_v[row, pl.ds(d * 16, 16)] = acc[d]
            return 0

        lax.fori_loop(0, MB, pool_row, 0)

    pltpu.sync_copy(out_v, out_hbm.at[pl.ds(wid * BPW, BPW)])


def kernel(x, table):
    xr = x.astype(jnp.int32).reshape(NW, MC, NSUB, SUBW)
    return _field_embed(xr, table)
